# baseline (speedup 1.0000x reference)
import functools

import jax
import jax.numpy as jnp
from jax.experimental import pallas as pl

_NUM_EXPERTS = 16
_TOP_K = 2
_BLOCK_ROWS = 512


def _gate_kernel(x_ref, w_ref, idx_ref, val_ref):
    x = x_ref[...]
    w = w_ref[...]
    logits = jax.lax.dot_general(
        x, w, (((1,), (1,)), ((), ())), preferred_element_type=jnp.float32
    )

    col = jax.lax.broadcasted_iota(jnp.int32, logits.shape, 1)

    m1 = jnp.max(logits, axis=1, keepdims=True)
    i1 = jnp.min(jnp.where(logits == m1, col, _NUM_EXPERTS), axis=1)

    masked = jnp.where(col == i1[:, None], -jnp.inf, logits)
    m2 = jnp.max(masked, axis=1, keepdims=True)
    i2 = jnp.min(jnp.where(masked == m2, col, _NUM_EXPERTS), axis=1)

    ex = jnp.exp(logits - m1)
    denom = jnp.sum(ex, axis=1, keepdims=True)
    v1 = 1.0 / denom[:, 0]
    v2 = jnp.exp(m2 - m1)[:, 0] / denom[:, 0]

    idx_ref[...] = jnp.concatenate([i1[:, None], i2[:, None]], axis=1)
    val_ref[...] = jnp.concatenate([v1[:, None], v2[:, None]], axis=1)


@jax.jit
def kernel(hidden_states, weight):
    d = hidden_states.shape[-1]
    hs = hidden_states.reshape(-1, d)
    t = hs.shape[0]
    grid = (t // _BLOCK_ROWS,)

    idx, val = pl.pallas_call(
        _gate_kernel,
        grid=grid,
        in_specs=[
            pl.BlockSpec((_BLOCK_ROWS, d), lambda i: (i, 0)),
            pl.BlockSpec((_NUM_EXPERTS, d), lambda i: (0, 0)),
        ],
        out_specs=[
            pl.BlockSpec((_BLOCK_ROWS, _TOP_K), lambda i: (i, 0)),
            pl.BlockSpec((_BLOCK_ROWS, _TOP_K), lambda i: (i, 0)),
        ],
        out_shape=[
            jax.ShapeDtypeStruct((t, _TOP_K), jnp.int32),
            jax.ShapeDtypeStruct((t, _TOP_K), jnp.float32),
        ],
    )(hs, weight)
    return idx, val


# speedup vs baseline: 1.1351x; 1.1351x over previous
import functools

import jax
import jax.numpy as jnp
from jax.experimental import pallas as pl

_NUM_EXPERTS = 16
_TOP_K = 2
_BLOCK_ROWS = 1024


def _gate_kernel(x_ref, w_ref, idx_ref, val_ref):
    x = x_ref[...]
    w = w_ref[...]
    logits = jax.lax.dot_general(
        x, w, (((1,), (1,)), ((), ())), preferred_element_type=jnp.float32
    )

    col = jax.lax.broadcasted_iota(jnp.int32, logits.shape, 1)

    m1 = jnp.max(logits, axis=1, keepdims=True)
    i1 = jnp.min(jnp.where(logits == m1, col, _NUM_EXPERTS), axis=1)

    masked = jnp.where(col == i1[:, None], -jnp.inf, logits)
    m2 = jnp.max(masked, axis=1, keepdims=True)
    i2 = jnp.min(jnp.where(masked == m2, col, _NUM_EXPERTS), axis=1)

    ex = jnp.exp(logits - m1)
    denom = jnp.sum(ex, axis=1, keepdims=True)
    v1 = 1.0 / denom[:, 0]
    v2 = jnp.exp(m2 - m1)[:, 0] / denom[:, 0]

    idx_ref[...] = jnp.concatenate([i1[:, None], i2[:, None]], axis=1)
    val_ref[...] = jnp.concatenate([v1[:, None], v2[:, None]], axis=1)


@jax.jit
def kernel(hidden_states, weight):
    d = hidden_states.shape[-1]
    hs = hidden_states.reshape(-1, d)
    t = hs.shape[0]
    grid = (t // _BLOCK_ROWS,)

    idx, val = pl.pallas_call(
        _gate_kernel,
        grid=grid,
        in_specs=[
            pl.BlockSpec((_BLOCK_ROWS, d), lambda i: (i, 0)),
            pl.BlockSpec((_NUM_EXPERTS, d), lambda i: (0, 0)),
        ],
        out_specs=[
            pl.BlockSpec((_BLOCK_ROWS, _TOP_K), lambda i: (i, 0)),
            pl.BlockSpec((_BLOCK_ROWS, _TOP_K), lambda i: (i, 0)),
        ],
        out_shape=[
            jax.ShapeDtypeStruct((t, _TOP_K), jnp.int32),
            jax.ShapeDtypeStruct((t, _TOP_K), jnp.float32),
        ],
    )(hs, weight)
    return idx, val


# speedup vs baseline: 1.1510x; 1.0140x over previous
import functools

import jax
import jax.numpy as jnp
from jax.experimental import pallas as pl

_NUM_EXPERTS = 16
_TOP_K = 2
_BLOCK_ROWS = 2048


def _gate_kernel(x_ref, w_ref, idx_ref, val_ref):
    x = x_ref[...]
    w = w_ref[...]
    logits = jax.lax.dot_general(
        x, w, (((1,), (1,)), ((), ())), preferred_element_type=jnp.float32
    )

    col = jax.lax.broadcasted_iota(jnp.int32, logits.shape, 1)

    m1 = jnp.max(logits, axis=1, keepdims=True)
    i1 = jnp.min(jnp.where(logits == m1, col, _NUM_EXPERTS), axis=1)

    masked = jnp.where(col == i1[:, None], -jnp.inf, logits)
    m2 = jnp.max(masked, axis=1, keepdims=True)
    i2 = jnp.min(jnp.where(masked == m2, col, _NUM_EXPERTS), axis=1)

    ex = jnp.exp(logits - m1)
    denom = jnp.sum(ex, axis=1, keepdims=True)
    v1 = 1.0 / denom[:, 0]
    v2 = jnp.exp(m2 - m1)[:, 0] / denom[:, 0]

    idx_ref[...] = jnp.concatenate([i1[:, None], i2[:, None]], axis=1)
    val_ref[...] = jnp.concatenate([v1[:, None], v2[:, None]], axis=1)


@jax.jit
def kernel(hidden_states, weight):
    d = hidden_states.shape[-1]
    hs = hidden_states.reshape(-1, d)
    t = hs.shape[0]
    grid = (t // _BLOCK_ROWS,)

    idx, val = pl.pallas_call(
        _gate_kernel,
        grid=grid,
        in_specs=[
            pl.BlockSpec((_BLOCK_ROWS, d), lambda i: (i, 0)),
            pl.BlockSpec((_NUM_EXPERTS, d), lambda i: (0, 0)),
        ],
        out_specs=[
            pl.BlockSpec((_BLOCK_ROWS, _TOP_K), lambda i: (i, 0)),
            pl.BlockSpec((_BLOCK_ROWS, _TOP_K), lambda i: (i, 0)),
        ],
        out_shape=[
            jax.ShapeDtypeStruct((t, _TOP_K), jnp.int32),
            jax.ShapeDtypeStruct((t, _TOP_K), jnp.float32),
        ],
    )(hs, weight)
    return idx, val
